# baseline (speedup 1.0000x reference)
import math

import jax
import jax.numpy as jnp
from jax.experimental import pallas as pl
from jax.experimental.pallas import tpu as pltpu

_V = 100000
_N = 1024
_PAD = 0
_SMOOTH = 0.1
_CONF = 1.0 - _SMOOTH
_FILL = _SMOOTH / (_V - 2)
_C = _FILL * math.log(_FILL) * (_V - 2) + _CONF * math.log(_CONF)

_VB = 2048
_NVB = (_V + _VB - 1) // _VB

_GR = 8
_NGB = _N // _GR


def _stream_kernel(x_ref, tgt_ref, out_ref, acc_ref):
    k = pl.program_id(0)
    xb = x_ref[...]

    @pl.when(k == 0)
    def _init():
        acc_ref[...] = _FILL * xb[:, 0:1]

    @pl.when(k < _NVB - 1)
    def _body():
        acc_ref[...] += -_FILL * jnp.sum(xb, axis=1, keepdims=True)

    @pl.when(k == _NVB - 1)
    def _last():
        col = jax.lax.broadcasted_iota(jnp.int32, (_N, _VB), 1) + k * _VB
        xm = jnp.where(col < _V, xb, 0.0)
        acc = acc_ref[...] - _FILL * jnp.sum(xm, axis=1, keepdims=True)
        valid = tgt_ref[...] != _PAD
        out_ref[0, 0] = jnp.sum(jnp.where(valid, acc + _C, 0.0))


def _make_gather_spec(j):
    return pl.BlockSpec((_GR, 128), lambda i, tgt: (i, tgt[i * _GR + j] // 128))


def _gather_kernel(tgt_sm, *refs):
    i = pl.program_id(0)
    out_ref = refs[-1]
    lane = jax.lax.broadcasted_iota(jnp.int32, (_GR, 128), 1)
    sub = jax.lax.broadcasted_iota(jnp.int32, (_GR, 128), 0)
    s = jnp.float32(0.0)
    for j in range(_GR):
        t = tgt_sm[i * _GR + j]
        sel = (sub == j) & (lane == t % 128)
        v = jnp.sum(jnp.where(sel, refs[j][...], 0.0))
        s += jnp.where(t != _PAD, v, 0.0)

    @pl.when(i == 0)
    def _init():
        out_ref[0, 0] = 0.0

    out_ref[0, 0] += (_FILL - _CONF) * s


def kernel(x, target):
    tgt2 = target.reshape(_N, 1)
    s1 = pl.pallas_call(
        _stream_kernel,
        grid=(_NVB,),
        in_specs=[
            pl.BlockSpec((_N, _VB), lambda k: (0, k)),
            pl.BlockSpec((_N, 1), lambda k: (0, 0)),
        ],
        out_specs=pl.BlockSpec((1, 1), lambda k: (0, 0),
                               memory_space=pltpu.SMEM),
        out_shape=jax.ShapeDtypeStruct((1, 1), jnp.float32),
        scratch_shapes=[pltpu.VMEM((_N, 1), jnp.float32)],
    )(x, tgt2)

    s2 = pl.pallas_call(
        _gather_kernel,
        grid_spec=pltpu.PrefetchScalarGridSpec(
            num_scalar_prefetch=1,
            grid=(_NGB,),
            in_specs=[_make_gather_spec(j) for j in range(_GR)],
            out_specs=pl.BlockSpec((1, 1), lambda i, tgt: (0, 0),
                                   memory_space=pltpu.SMEM),
        ),
        out_shape=jax.ShapeDtypeStruct((1, 1), jnp.float32),
    )(target, *([x] * _GR))

    return s1[0, 0] + s2[0, 0]


# speedup vs baseline: 1.0118x; 1.0118x over previous
import math

import jax
import jax.numpy as jnp
from jax.experimental import pallas as pl
from jax.experimental.pallas import tpu as pltpu

_V = 100000
_N = 1024
_PAD = 0
_SMOOTH = 0.1
_CONF = 1.0 - _SMOOTH
_FILL = _SMOOTH / (_V - 2)
_C = _FILL * math.log(_FILL) * (_V - 2) + _CONF * math.log(_CONF)

_RB = 32
_NRB = _N // _RB

_GR = 8
_NGB = _N // _GR


def _stream_kernel(x_ref, tgt_ref, out_ref):
    k = pl.program_id(0)
    xb = x_ref[...]
    rowsum = jnp.sum(xb, axis=1, keepdims=True)
    x0 = xb[:, 0:1]
    valid = tgt_ref[...] != _PAD
    s = jnp.sum(jnp.where(valid, _C - _FILL * (rowsum - x0), 0.0))

    @pl.when(k == 0)
    def _init():
        out_ref[0, 0] = 0.0

    out_ref[0, 0] += s


def _make_gather_spec(j):
    return pl.BlockSpec((_GR, 128), lambda i, tgt: (i, tgt[i * _GR + j] // 128))


def _gather_kernel(tgt_sm, *refs):
    i = pl.program_id(0)
    out_ref = refs[-1]
    lane = jax.lax.broadcasted_iota(jnp.int32, (_GR, 128), 1)
    sub = jax.lax.broadcasted_iota(jnp.int32, (_GR, 128), 0)
    s = jnp.float32(0.0)
    for j in range(_GR):
        t = tgt_sm[i * _GR + j]
        sel = (sub == j) & (lane == t % 128)
        v = jnp.sum(jnp.where(sel, refs[j][...], 0.0))
        s += jnp.where(t != _PAD, v, 0.0)

    @pl.when(i == 0)
    def _init():
        out_ref[0, 0] = 0.0

    out_ref[0, 0] += (_FILL - _CONF) * s


def kernel(x, target):
    tgt2 = target.reshape(_N, 1)
    s1 = pl.pallas_call(
        _stream_kernel,
        grid=(_NRB,),
        in_specs=[
            pl.BlockSpec((_RB, _V), lambda k: (k, 0)),
            pl.BlockSpec((_RB, 1), lambda k: (k, 0)),
        ],
        out_specs=pl.BlockSpec((1, 1), lambda k: (0, 0),
                               memory_space=pltpu.SMEM),
        out_shape=jax.ShapeDtypeStruct((1, 1), jnp.float32),
    )(x, tgt2)

    s2 = pl.pallas_call(
        _gather_kernel,
        grid_spec=pltpu.PrefetchScalarGridSpec(
            num_scalar_prefetch=1,
            grid=(_NGB,),
            in_specs=[_make_gather_spec(j) for j in range(_GR)],
            out_specs=pl.BlockSpec((1, 1), lambda i, tgt: (0, 0),
                                   memory_space=pltpu.SMEM),
        ),
        out_shape=jax.ShapeDtypeStruct((1, 1), jnp.float32),
    )(target, *([x] * _GR))

    return s1[0, 0] + s2[0, 0]
